# baseline (speedup 1.0000x reference)
import functools

import jax
import jax.numpy as jnp
from jax import lax
from jax.experimental import pallas as pl
from jax.experimental.pallas import tpu as pltpu
from jax.experimental.pallas import tpu_sc as plsc

NUM_EMBEDDINGS = 1000000
EMBEDDING_DIM = 32

_INFO = plsc.get_sparse_core_info()
_NC, _NS = _INFO.num_cores, _INFO.num_subcores
_NW = _NC * _NS

_ROWS = 16384
_T = 26
_TP = 32
_EP = 64
_B = _ROWS * _T
_LANE = 128
_IDX_ROWS = _B // _LANE
_IRPW = _IDX_ROWS // _NW
_RPW = _ROWS // _NW
_NBUF = 2


def _make_kernel():
  mesh = plsc.VectorSubcoreMesh(core_axis_name="c", subcore_axis_name="s")

  @functools.partial(
      pl.kernel,
      out_type=jax.ShapeDtypeStruct((_ROWS, _TP, _EP), jnp.float32),
      mesh=mesh,
      scratch_types=(
          [pltpu.VMEM((_IRPW, _LANE), jnp.int32),
           pltpu.VMEM((_T, _RPW), jnp.int32)]
          + [pltpu.VMEM((_RPW, EMBEDDING_DIM), jnp.float32)] * _NBUF
          + [pltpu.SemaphoreType.DMA] * (2 * _NBUF)
      ),
      compiler_params=pltpu.CompilerParams(
          use_tc_tiling_on_sc=False, needs_layout_passes=False
      ),
  )
  def emb_kernel(idx_hbm, table_hbm, out_hbm, idx_v, idx_t, *scratch):
    rows = scratch[:_NBUF]
    gsem = scratch[_NBUF:2 * _NBUF]
    osem = scratch[2 * _NBUF:]
    wid = lax.axis_index("s") * _NC + lax.axis_index("c")
    rbase = wid * _RPW
    pltpu.sync_copy(idx_hbm.at[pl.ds(wid * _IRPW, _IRPW)], idx_v)

    @pl.loop(0, _T)
    def _tr(t):
      for k in range(_RPW // 16):
        f = (lax.iota(jnp.int32, 16) + k * 16) * _T + t
        i0 = lax.shift_right_logical(f, 7)
        i1 = lax.bitwise_and(f, 127)
        idx_t[t, pl.ds(k * 16, 16)] = plsc.load_gather(idx_v, [i0, i1])

    def start_gather(t, b):
      pltpu.async_copy(table_hbm.at[idx_t.at[t]], rows[b], gsem[b])

    def wait_gather(b):
      pltpu.make_async_copy(table_hbm.at[idx_t.at[0]], rows[b], gsem[b]).wait()

    for b in range(_NBUF):
      start_gather(b, b)

    @pl.loop(0, _T)
    def _pos(t):
      b0 = lax.rem(t, _NBUF)
      for b in range(_NBUF):

        @pl.when(b0 == b)
        def _():
          wait_gather(b)
          pltpu.async_copy(
              rows[b],
              out_hbm.at[pl.ds(rbase, _RPW), t, pl.ds(0, EMBEDDING_DIM)],
              osem[b],
          )
          pltpu.make_async_copy(
              rows[b],
              out_hbm.at[pl.ds(rbase, _RPW), 0, pl.ds(0, EMBEDDING_DIM)],
              osem[b],
          ).wait()

          @pl.when(t < _T - _NBUF)
          def _():
            start_gather(t + _NBUF, b)

  return emb_kernel


_EMB = _make_kernel()


@jax.jit
def kernel(token_ids, E):
  flat = token_ids.reshape(_IDX_ROWS, _LANE)
  out = _EMB(flat, E)
  return out[:, :_T, :EMBEDDING_DIM]


# speedup vs baseline: 1.0480x; 1.0480x over previous
import functools

import jax
import jax.numpy as jnp
from jax import lax
from jax.experimental import pallas as pl
from jax.experimental.pallas import tpu as pltpu
from jax.experimental.pallas import tpu_sc as plsc

NUM_EMBEDDINGS = 1000000
EMBEDDING_DIM = 32

_INFO = plsc.get_sparse_core_info()
_NC, _NS = _INFO.num_cores, _INFO.num_subcores
_NW = _NC * _NS

_B = 16384 * 26
_BPW = _B // _NW
_CHUNK = 832
_NCHUNK = _BPW // _CHUNK
_NBUF = 4
_NGROUP = _NCHUNK // _NBUF


def _make_kernel():
  mesh = plsc.VectorSubcoreMesh(core_axis_name="c", subcore_axis_name="s")

  @functools.partial(
      pl.kernel,
      out_type=jax.ShapeDtypeStruct((_B, EMBEDDING_DIM), jnp.float32),
      mesh=mesh,
      scratch_types=(
          [pltpu.VMEM((_BPW,), jnp.int32)]
          + [pltpu.VMEM((_CHUNK, EMBEDDING_DIM), jnp.float32)] * _NBUF
          + [pltpu.SemaphoreType.DMA] * (2 * _NBUF)
      ),
      compiler_params=pltpu.CompilerParams(use_tc_tiling_on_sc=False),
  )
  def emb_kernel(idx_hbm, table_hbm, out_hbm, idx_v, *scratch):
    rows = scratch[:_NBUF]
    gsem = scratch[_NBUF:2 * _NBUF]
    osem = scratch[2 * _NBUF:]
    wid = lax.axis_index("s") * _NC + lax.axis_index("c")
    base = wid * _BPW
    pltpu.sync_copy(idx_hbm.at[pl.ds(base, _BPW)], idx_v)

    def start_gather(c, b):
      pltpu.async_copy(
          table_hbm.at[idx_v.at[pl.ds(c * _CHUNK, _CHUNK)]], rows[b], gsem[b]
      )

    def wait_gather(b):
      pltpu.make_async_copy(
          table_hbm.at[idx_v.at[pl.ds(0, _CHUNK)]], rows[b], gsem[b]
      ).wait()

    for b in range(_NBUF):
      start_gather(b, b)

    @pl.loop(0, _NGROUP)
    def _group(g):
      for b in range(_NBUF):
        c = g * _NBUF + b
        wait_gather(b)
        pltpu.async_copy(
            rows[b], out_hbm.at[pl.ds(base + c * _CHUNK, _CHUNK)], osem[b]
        )
        pltpu.make_async_copy(
            rows[b], out_hbm.at[pl.ds(base, _CHUNK)], osem[b]
        ).wait()

        @pl.when(g < _NGROUP - 1)
        def _():
          start_gather(c + _NBUF, b)

  return emb_kernel


_EMB = _make_kernel()


@jax.jit
def kernel(token_ids, E):
  flat = token_ids.reshape(-1).astype(jnp.int32)
  out = _EMB(flat, E)
  return out.reshape(token_ids.shape + (EMBEDDING_DIM,))
